# Initial kernel scaffold; baseline (speedup 1.0000x reference)
import jax
import jax.numpy as jnp
from jax.experimental import pallas as pl


def kernel(features_list, adj_mats_list, W, a_src, a_dst):
    raise NotImplementedError("write your pallas kernel here")


# speedup vs baseline: 9446.2941x; 9446.2941x over previous
import jax
import jax.numpy as jnp
from jax.experimental import pallas as pl
from jax.experimental.pallas import tpu as pltpu

_NEG_SLOPE = 0.2
_MASK_VAL = -1e30


def _gat_kernel(adj_ref, x_ref, w_ref, asrc_ref, adst_ref,
                out_ref, lsum_ref, h_s, m_s, l_s, acc_s, *, bi, bj):
    j = pl.program_id(1)
    i = pl.program_id(2)
    ni = pl.num_programs(2)

    @pl.when(jnp.logical_and(j == 0, i == 0))
    def _():
        h_s[...] = jnp.dot(x_ref[0], w_ref[...],
                           preferred_element_type=jnp.float32)

    @pl.when(i == 0)
    def _():
        m_s[...] = jnp.full_like(m_s, _MASK_VAL)
        l_s[...] = jnp.zeros_like(l_s)
        acc_s[...] = jnp.zeros_like(acc_s)

    adj = adj_ref[0]
    h_i = h_s[pl.ds(i * bi, bi), :]
    h_j = h_s[pl.ds(j * bj, bj), :]

    s_col = jnp.dot(h_i, asrc_ref[...],
                    preferred_element_type=jnp.float32)
    t_row = jax.lax.dot_general(adst_ref[...], h_j,
                                (((1,), (1,)), ((), ())),
                                preferred_element_type=jnp.float32)

    e = s_col + t_row
    e = jnp.where(e >= 0, e, _NEG_SLOPE * e)
    mask = adj != 0
    e_m = jnp.where(mask, e, _MASK_VAL)

    m_old = m_s[...]
    m_new = jnp.maximum(m_old, jnp.max(e_m, axis=0, keepdims=True))
    scale = jnp.exp(m_old - m_new)
    p = jnp.where(mask, jnp.exp(e - m_new), 0.0)

    m_s[...] = m_new
    l_s[...] = l_s[...] * scale + jnp.sum(p, axis=0, keepdims=True)
    acc_s[...] = acc_s[...] * scale + jax.lax.dot_general(
        h_i, p, (((0,), (0,)), ((), ())),
        preferred_element_type=jnp.float32)

    @pl.when(i == ni - 1)
    def _():
        l = l_s[...]
        r = acc_s[...] / (l + 1e-16)
        res = jnp.where(r > 0, r, jnp.exp(jnp.minimum(r, 0.0)) - 1.0)
        out_ref[0] = res.T
        lsum_ref[0] = l


def kernel(features_list, adj_mats_list, W, a_src, a_dst):
    G, N, D_in = features_list.shape
    D_out = W.shape[1]
    bi = min(512, N)
    bj = min(1024, N)
    grid = (G, N // bj, N // bi)

    asrc = a_src.reshape(D_out, 1)
    adst = a_dst.reshape(1, D_out)

    out, lsum = pl.pallas_call(
        lambda *refs: _gat_kernel(*refs, bi=bi, bj=bj),
        grid=grid,
        in_specs=[
            pl.BlockSpec((1, bi, bj), lambda g, j, i: (g, i, j)),
            pl.BlockSpec((1, N, D_in), lambda g, j, i: (g, 0, 0)),
            pl.BlockSpec((D_in, D_out), lambda g, j, i: (0, 0)),
            pl.BlockSpec((D_out, 1), lambda g, j, i: (0, 0)),
            pl.BlockSpec((1, D_out), lambda g, j, i: (0, 0)),
        ],
        out_specs=[
            pl.BlockSpec((1, bj, D_out), lambda g, j, i: (g, j, 0)),
            pl.BlockSpec((1, 1, bj), lambda g, j, i: (g, 0, j)),
        ],
        out_shape=[
            jax.ShapeDtypeStruct((G, N, D_out), jnp.float32),
            jax.ShapeDtypeStruct((G, 1, N), jnp.float32),
        ],
        scratch_shapes=[
            pltpu.VMEM((N, D_out), jnp.float32),
            pltpu.VMEM((1, bj), jnp.float32),
            pltpu.VMEM((1, bj), jnp.float32),
            pltpu.VMEM((D_out, bj), jnp.float32),
        ],
    )(adj_mats_list, features_list, W, asrc, adst)

    has_edges = jnp.max(lsum, axis=(1, 2)) > 0.0
    res = jnp.where(has_edges[:, None, None], out, features_list)
    return tuple(res[g] for g in range(G))


# speedup vs baseline: 10709.6327x; 1.1337x over previous
import jax
import jax.numpy as jnp
from jax.experimental import pallas as pl
from jax.experimental.pallas import tpu as pltpu

_NEG_SLOPE = 0.2


def _gat_kernel(adj_ref, x_ref, w_ref, asrc_ref, adst_ref,
                out_ref, lsum_ref, h_s, l_s, acc_s, *, bi, bj):
    j = pl.program_id(1)
    i = pl.program_id(2)
    ni = pl.num_programs(2)

    @pl.when(jnp.logical_and(j == 0, i == 0))
    def _():
        h_s[...] = jnp.dot(x_ref[0], w_ref[...],
                           preferred_element_type=jnp.float32)

    @pl.when(i == 0)
    def _():
        l_s[...] = jnp.zeros_like(l_s)
        acc_s[...] = jnp.zeros_like(acc_s)

    adj = adj_ref[0]
    h_i = h_s[pl.ds(i * bi, bi), :]
    h_j = h_s[pl.ds(j * bj, bj), :]

    s_col = jnp.dot(h_i, asrc_ref[...],
                    preferred_element_type=jnp.float32)
    t_row = jax.lax.dot_general(adst_ref[...], h_j,
                                (((1,), (1,)), ((), ())),
                                preferred_element_type=jnp.float32)

    e = s_col + t_row
    e = jnp.maximum(e, _NEG_SLOPE * e)
    p = adj * jnp.exp(e)

    l_s[...] = l_s[...] + jnp.sum(p, axis=0, keepdims=True)
    acc_s[...] = acc_s[...] + jax.lax.dot_general(
        h_i, p, (((0,), (0,)), ((), ())),
        preferred_element_type=jnp.float32)

    @pl.when(i == ni - 1)
    def _():
        l = l_s[...]
        r = acc_s[...] / (l + 1e-16)
        res = jnp.where(r > 0, r, jnp.exp(jnp.minimum(r, 0.0)) - 1.0)
        out_ref[0] = res.T
        lsum_ref[0] = l


def kernel(features_list, adj_mats_list, W, a_src, a_dst):
    G, N, D_in = features_list.shape
    D_out = W.shape[1]
    bi = min(512, N)
    bj = min(1024, N)
    grid = (G, N // bj, N // bi)

    asrc = a_src.reshape(D_out, 1)
    adst = a_dst.reshape(1, D_out)

    out, lsum = pl.pallas_call(
        lambda *refs: _gat_kernel(*refs, bi=bi, bj=bj),
        grid=grid,
        in_specs=[
            pl.BlockSpec((1, bi, bj), lambda g, j, i: (g, i, j)),
            pl.BlockSpec((1, N, D_in), lambda g, j, i: (g, 0, 0)),
            pl.BlockSpec((D_in, D_out), lambda g, j, i: (0, 0)),
            pl.BlockSpec((D_out, 1), lambda g, j, i: (0, 0)),
            pl.BlockSpec((1, D_out), lambda g, j, i: (0, 0)),
        ],
        out_specs=[
            pl.BlockSpec((1, bj, D_out), lambda g, j, i: (g, j, 0)),
            pl.BlockSpec((1, 1, bj), lambda g, j, i: (g, 0, j)),
        ],
        out_shape=[
            jax.ShapeDtypeStruct((G, N, D_out), jnp.float32),
            jax.ShapeDtypeStruct((G, 1, N), jnp.float32),
        ],
        scratch_shapes=[
            pltpu.VMEM((N, D_out), jnp.float32),
            pltpu.VMEM((1, bj), jnp.float32),
            pltpu.VMEM((D_out, bj), jnp.float32),
        ],
    )(adj_mats_list, features_list, W, asrc, adst)

    has_edges = jnp.max(lsum, axis=(1, 2)) > 0.0
    res = jnp.where(has_edges[:, None, None], out, features_list)
    return tuple(res[g] for g in range(G))


# speedup vs baseline: 16585.6441x; 1.5487x over previous
import jax
import jax.numpy as jnp
from jax.experimental import pallas as pl
from jax.experimental.pallas import tpu as pltpu

_NEG_SLOPE = 0.2


def _gat_kernel(adj_ref, x_ref, w_ref, asrc_ref, adst_ref,
                out_ref, lsum_ref, h_s, l_s, acc_s, *, bi, bj):
    j = pl.program_id(1)
    i = pl.program_id(2)
    ni = pl.num_programs(2)

    @pl.when(jnp.logical_and(j == 0, i == 0))
    def _():
        h_s[...] = jnp.dot(x_ref[0], w_ref[...],
                           preferred_element_type=jnp.float32)

    @pl.when(i == 0)
    def _():
        l_s[...] = jnp.zeros_like(l_s)
        acc_s[...] = jnp.zeros_like(acc_s)

    adj = adj_ref[0]
    h_i = h_s[pl.ds(i * bi, bi), :]
    h_j = h_s[pl.ds(j * bj, bj), :]

    s_col = jnp.dot(h_i, asrc_ref[...],
                    preferred_element_type=jnp.float32)
    t_row = jax.lax.dot_general(adst_ref[...], h_j,
                                (((1,), (1,)), ((), ())),
                                preferred_element_type=jnp.float32)

    e = s_col + t_row
    e = jnp.maximum(e, _NEG_SLOPE * e)
    p = adj * jnp.exp(e)

    l_s[...] = l_s[...] + jnp.sum(p, axis=0, keepdims=True)
    acc_s[...] = acc_s[...] + jax.lax.dot_general(
        h_i, p, (((0,), (0,)), ((), ())),
        preferred_element_type=jnp.float32)

    @pl.when(i == ni - 1)
    def _():
        l = l_s[...]
        r = acc_s[...] / (l + 1e-16)
        res = jnp.where(r > 0, r, jnp.exp(jnp.minimum(r, 0.0)) - 1.0)
        out_ref[0] = res.T
        lsum_ref[0] = l


def kernel(features_list, adj_mats_list, W, a_src, a_dst):
    G, N, D_in = features_list.shape
    D_out = W.shape[1]
    bi = min(512, N)
    bj = min(4096, N)
    grid = (G, N // bj, N // bi)

    asrc = a_src.reshape(D_out, 1)
    adst = a_dst.reshape(1, D_out)

    out, lsum = pl.pallas_call(
        lambda *refs: _gat_kernel(*refs, bi=bi, bj=bj),
        grid=grid,
        in_specs=[
            pl.BlockSpec((1, bi, bj), lambda g, j, i: (g, i, j)),
            pl.BlockSpec((1, N, D_in), lambda g, j, i: (g, 0, 0)),
            pl.BlockSpec((D_in, D_out), lambda g, j, i: (0, 0)),
            pl.BlockSpec((D_out, 1), lambda g, j, i: (0, 0)),
            pl.BlockSpec((1, D_out), lambda g, j, i: (0, 0)),
        ],
        out_specs=[
            pl.BlockSpec((1, bj, D_out), lambda g, j, i: (g, j, 0)),
            pl.BlockSpec((1, 1, bj), lambda g, j, i: (g, 0, j)),
        ],
        out_shape=[
            jax.ShapeDtypeStruct((G, N, D_out), jnp.float32),
            jax.ShapeDtypeStruct((G, 1, N), jnp.float32),
        ],
        scratch_shapes=[
            pltpu.VMEM((N, D_out), jnp.float32),
            pltpu.VMEM((1, bj), jnp.float32),
            pltpu.VMEM((D_out, bj), jnp.float32),
        ],
    )(adj_mats_list, features_list, W, asrc, adst)

    has_edges = jnp.max(lsum, axis=(1, 2)) > 0.0
    res = jnp.where(has_edges[:, None, None], out, features_list)
    return tuple(res[g] for g in range(G))


# speedup vs baseline: 17612.9607x; 1.0619x over previous
import jax
import jax.numpy as jnp
from jax.experimental import pallas as pl
from jax.experimental.pallas import tpu as pltpu

_NEG_SLOPE = 0.2


def _gat_kernel(adj_ref, x_ref, w_ref, asrc_ref, adst_ref,
                out_ref, lsum_ref, h_s, l_s, acc_s, *, bi, bj):
    j = pl.program_id(1)
    i = pl.program_id(2)
    ni = pl.num_programs(2)

    @pl.when(jnp.logical_and(j == 0, i == 0))
    def _():
        h_s[...] = jnp.dot(x_ref[0], w_ref[...],
                           preferred_element_type=jnp.float32)

    @pl.when(i == 0)
    def _():
        l_s[...] = jnp.zeros_like(l_s)
        acc_s[...] = jnp.zeros_like(acc_s)

    adj = adj_ref[0]
    h_i = h_s[pl.ds(i * bi, bi), :]
    h_j = h_s[pl.ds(j * bj, bj), :]

    s_col = jnp.dot(h_i, asrc_ref[...],
                    preferred_element_type=jnp.float32)
    t_row = jax.lax.dot_general(adst_ref[...], h_j,
                                (((1,), (1,)), ((), ())),
                                preferred_element_type=jnp.float32)

    e = s_col + t_row
    e = jnp.maximum(e, _NEG_SLOPE * e)
    p = adj * jnp.exp(e)

    l_s[...] = l_s[...] + jnp.sum(p, axis=0, keepdims=True)
    acc_s[...] = acc_s[...] + jax.lax.dot_general(
        h_i, p, (((0,), (0,)), ((), ())),
        preferred_element_type=jnp.float32)

    @pl.when(i == ni - 1)
    def _():
        l = l_s[...]
        r = acc_s[...] / (l + 1e-16)
        res = jnp.where(r > 0, r, jnp.exp(jnp.minimum(r, 0.0)) - 1.0)
        out_ref[0] = res.T
        lsum_ref[0] = l


def kernel(features_list, adj_mats_list, W, a_src, a_dst):
    G, N, D_in = features_list.shape
    D_out = W.shape[1]
    bi = min(1024, N)
    bj = min(4096, N)
    grid = (G, N // bj, N // bi)

    asrc = a_src.reshape(D_out, 1)
    adst = a_dst.reshape(1, D_out)

    out, lsum = pl.pallas_call(
        lambda *refs: _gat_kernel(*refs, bi=bi, bj=bj),
        grid=grid,
        in_specs=[
            pl.BlockSpec((1, bi, bj), lambda g, j, i: (g, i, j)),
            pl.BlockSpec((1, N, D_in), lambda g, j, i: (g, 0, 0)),
            pl.BlockSpec((D_in, D_out), lambda g, j, i: (0, 0)),
            pl.BlockSpec((D_out, 1), lambda g, j, i: (0, 0)),
            pl.BlockSpec((1, D_out), lambda g, j, i: (0, 0)),
        ],
        out_specs=[
            pl.BlockSpec((1, bj, D_out), lambda g, j, i: (g, j, 0)),
            pl.BlockSpec((1, 1, bj), lambda g, j, i: (g, 0, j)),
        ],
        out_shape=[
            jax.ShapeDtypeStruct((G, N, D_out), jnp.float32),
            jax.ShapeDtypeStruct((G, 1, N), jnp.float32),
        ],
        scratch_shapes=[
            pltpu.VMEM((N, D_out), jnp.float32),
            pltpu.VMEM((1, bj), jnp.float32),
            pltpu.VMEM((D_out, bj), jnp.float32),
        ],
    )(adj_mats_list, features_list, W, asrc, adst)

    has_edges = jnp.max(lsum, axis=(1, 2)) > 0.0
    res = jnp.where(has_edges[:, None, None], out, features_list)
    return tuple(res[g] for g in range(G))


# speedup vs baseline: 17953.0346x; 1.0193x over previous
import jax
import jax.numpy as jnp
from jax.experimental import pallas as pl
from jax.experimental.pallas import tpu as pltpu

_NEG_SLOPE = 0.2


def _gat_kernel(adj_ref, x_ref, w_ref, asrc_ref, adst_ref,
                out_ref, lsum_ref, h_s, acc_s, *, bi, bj, d_out):
    j = pl.program_id(1)
    i = pl.program_id(2)
    ni = pl.num_programs(2)
    dp = d_out + 8

    @pl.when(jnp.logical_and(j == 0, i == 0))
    def _():
        h_s[:, :d_out] = jnp.dot(x_ref[0], w_ref[...],
                                 preferred_element_type=jnp.float32)
        col = jax.lax.broadcasted_iota(jnp.int32, (h_s.shape[0], 8), 1)
        h_s[:, d_out:] = jnp.where(col == 0, 1.0, 0.0)

    @pl.when(i == 0)
    def _():
        acc_s[...] = jnp.zeros_like(acc_s)

    adj = adj_ref[0]
    h_i = h_s[pl.ds(i * bi, bi), :]
    h_j = h_s[pl.ds(j * bj, bj), :d_out]

    s_col = jnp.dot(h_i[:, :d_out], asrc_ref[...],
                    preferred_element_type=jnp.float32)
    t_row = jax.lax.dot_general(adst_ref[...], h_j,
                                (((1,), (1,)), ((), ())),
                                preferred_element_type=jnp.float32)

    e = s_col + t_row
    e = jnp.maximum(e, _NEG_SLOPE * e)
    p = adj * jnp.exp(e)

    acc_s[...] = acc_s[...] + jax.lax.dot_general(
        h_i, p, (((0,), (0,)), ((), ())),
        preferred_element_type=jnp.float32)

    @pl.when(i == ni - 1)
    def _():
        l = acc_s[pl.ds(d_out, 1), :]
        r = acc_s[:d_out, :] / (l + 1e-16)
        res = jnp.where(r > 0, r, jnp.exp(jnp.minimum(r, 0.0)) - 1.0)
        out_ref[0] = res.T
        lsum_ref[0] = l


def kernel(features_list, adj_mats_list, W, a_src, a_dst):
    G, N, D_in = features_list.shape
    D_out = W.shape[1]
    bi = min(1024, N)
    bj = min(4096, N)
    grid = (G, N // bj, N // bi)

    asrc = a_src.reshape(D_out, 1)
    adst = a_dst.reshape(1, D_out)

    out, lsum = pl.pallas_call(
        lambda *refs: _gat_kernel(*refs, bi=bi, bj=bj, d_out=D_out),
        grid=grid,
        in_specs=[
            pl.BlockSpec((1, bi, bj), lambda g, j, i: (g, i, j)),
            pl.BlockSpec((1, N, D_in), lambda g, j, i: (g, 0, 0)),
            pl.BlockSpec((D_in, D_out), lambda g, j, i: (0, 0)),
            pl.BlockSpec((D_out, 1), lambda g, j, i: (0, 0)),
            pl.BlockSpec((1, D_out), lambda g, j, i: (0, 0)),
        ],
        out_specs=[
            pl.BlockSpec((1, bj, D_out), lambda g, j, i: (g, j, 0)),
            pl.BlockSpec((1, 1, bj), lambda g, j, i: (g, 0, j)),
        ],
        out_shape=[
            jax.ShapeDtypeStruct((G, N, D_out), jnp.float32),
            jax.ShapeDtypeStruct((G, 1, N), jnp.float32),
        ],
        scratch_shapes=[
            pltpu.VMEM((N, D_out + 8), jnp.float32),
            pltpu.VMEM((D_out + 8, bj), jnp.float32),
        ],
    )(adj_mats_list, features_list, W, asrc, adst)

    has_edges = jnp.max(lsum, axis=(1, 2)) > 0.0
    res = jnp.where(has_edges[:, None, None], out, features_list)
    return tuple(res[g] for g in range(G))
